# Initial kernel scaffold; baseline (speedup 1.0000x reference)
import jax
import jax.numpy as jnp
from jax.experimental import pallas as pl


def kernel(updates, mask):
    raise NotImplementedError("write your pallas kernel here")


# speedup vs baseline: 4.7974x; 4.7974x over previous
import functools

import jax
import jax.numpy as jnp
from jax import lax
from jax.experimental import pallas as pl
from jax.experimental.pallas import tpu as pltpu
from jax.experimental.pallas import tpu_sc as plsc

B, H, W, C = 4, 112, 112, 192
OUT_H, OUT_W = H * 2, W * 2
TOTAL = B * OUT_H * OUT_W * C
N = B * H * W * C
CHUNK = 1_835_008
NCHUNK = TOTAL // CHUNK
NSUB = 16
PASSES = 11
PER_TILE = N // NSUB
WIN = 4096
WROWS = WIN // 128
NWIN = PER_TILE // WIN
C16 = CHUNK // NSUB
ALT_MASK = (1 << 20) - 1


def _body(idx_hbm, upd_hbm, zero_hbm, out_hbm, idxb, valb, addrb, acc):
    c = lax.axis_index("c")
    s = lax.axis_index("s")

    for p in range(PASSES):
        k = c * PASSES + p

        @pl.when(k < NCHUNK)
        def _pass():
            lo = k * CHUNK
            hi = lo + CHUNK

            pltpu.sync_copy(zero_hbm.at[pl.ds(s * C16, C16)],
                            acc.at[pl.ds(s * C16, C16)])
            plsc.subcore_barrier()

            def window(w, carry):
                base = s * PER_TILE + w * WIN
                pltpu.sync_copy(idx_hbm.at[pl.ds(base, WIN)], idxb)
                pltpu.sync_copy(upd_hbm.at[pl.ds(base, WIN)], valb)

                def vrow(r, carry2):
                    for cc in range(0, 128, 16):
                        off = r * 128 + cc
                        iv = idxb[pl.ds(off, 16)]
                        inr = (iv >= lo) & (iv < hi)
                        addr = jnp.where(inr, iv - lo, iv & ALT_MASK)
                        v = valb[pl.ds(off, 16)]
                        addrb[pl.ds(off, 16)] = addr
                        valb[pl.ds(off, 16)] = jnp.where(
                            inr, v, jnp.zeros((16,), jnp.float32))
                    return carry2

                lax.fori_loop(0, WROWS, vrow, 0, unroll=False)
                pltpu.sync_copy(valb, acc.at[addrb], add=True)
                return carry

            lax.fori_loop(0, NWIN, window, 0, unroll=False)
            plsc.subcore_barrier()

            pltpu.sync_copy(acc.at[pl.ds(s * C16, C16)],
                            out_hbm.at[pl.ds(lo + s * C16, C16)])


_scatter = functools.partial(
    pl.kernel,
    out_type=jax.ShapeDtypeStruct((TOTAL,), jnp.float32),
    mesh=plsc.VectorSubcoreMesh(core_axis_name="c", subcore_axis_name="s"),
    scratch_types=[
        pltpu.VMEM((WIN,), jnp.int32),
        pltpu.VMEM((WIN,), jnp.float32),
        pltpu.VMEM((WIN,), jnp.int32),
        pltpu.VMEM_SHARED((CHUNK,), jnp.float32),
    ],
)(_body)


@jax.jit
def kernel(updates, mask):
    idx = mask.astype(jnp.int32).reshape(N)
    upd = updates.reshape(N)
    zero = jnp.zeros((CHUNK,), jnp.float32)
    out = _scatter(idx, upd, zero)
    return out.reshape(-1, OUT_H, OUT_W, C)
